# Initial kernel scaffold; baseline (speedup 1.0000x reference)
import jax
import jax.numpy as jnp
from jax.experimental import pallas as pl


def kernel(x, edge_index, edge_attr, W1, b1, g1, be1, W2, b2, g2, be2, W3, b3, g3, be3, Wl, bl):
    raise NotImplementedError("write your pallas kernel here")


# speedup vs baseline: 32.7756x; 32.7756x over previous
import functools

import jax
import jax.numpy as jnp
from jax import lax
from jax.experimental import pallas as pl
from jax.experimental.pallas import tpu as pltpu
from jax.experimental.pallas import tpu_sc as plsc

N = 10000
E = 320000
NPAD = 10240
NC, NS, L = 2, 16, 16
NW = NC * NS
MB = 128
NB = 80
RING = 2
EPW = NB * MB
EPAD = EPW * NW
BNS = 0.9999950000374997


def _sc_agg(rows3, cols3, ew3, tab, zer, NF):
    scratch = (
        [pltpu.VMEM((NB, MB), jnp.int32)] * 2
        + [pltpu.VMEM((NB, MB), jnp.float32)]
        + [pltpu.VMEM((NF, MB), jnp.float32) for _ in range(2 * RING)]
        + [pltpu.VMEM_SHARED((NPAD,), jnp.float32) for _ in range(2 * NF)]
        + [pltpu.SemaphoreType.DMA] * (2 * RING)
    )

    @functools.partial(
        pl.kernel,
        out_type=jax.ShapeDtypeStruct((NC, 1, NF * NPAD), jnp.float32),
        mesh=plsc.VectorSubcoreMesh(core_axis_name="c", subcore_axis_name="s"),
        scratch_types=scratch,
    )
    def k(rows_h, cols_h, ew_h, tab_h, zer_h, out_h, *refs):
        idxr, idxc, ewv = refs[0], refs[1], refs[2]
        gval = list(refs[3:3 + RING])
        mval = list(refs[3 + RING:3 + 2 * RING])
        tabs = list(refs[3 + 2 * RING:3 + 2 * RING + NF])
        accs = list(refs[3 + 2 * RING + NF:3 + 2 * RING + 2 * NF])
        gsems = list(refs[3 + 2 * RING + 2 * NF:3 + 3 * RING + 2 * NF])
        ssems = list(refs[3 + 3 * RING + 2 * NF:])
        c = lax.axis_index("c")
        s = lax.axis_index("s")
        wid = s * NC + c
        pltpu.sync_copy(rows_h.at[wid], idxr)
        pltpu.sync_copy(cols_h.at[wid], idxc)
        pltpu.sync_copy(ew_h.at[wid], ewv)

        for f in range(NF):
            @pl.when(s == f)
            def _(f=f):
                pltpu.sync_copy(tab_h.at[pl.ds(f * NPAD, NPAD)], tabs[f])
                pltpu.sync_copy(zer_h.at[pl.ds(f * NPAD, NPAD)], accs[f])

        plsc.subcore_barrier()

        def fire_gathers(j, cc):
            for f in range(NF):
                pltpu.async_copy(tabs[f].at[idxr.at[j]], gval[cc].at[f],
                                 gsems[cc])

        def drain(sem, buf):
            for f in range(NF):
                pltpu.make_async_copy(zer_h.at[pl.ds(0, MB)], buf.at[f],
                                      sem).wait()

        for cc in range(RING):
            fire_gathers(cc, cc)

        def outer(jo, carry):
            for cc in range(RING):
                j = jo * RING + cc
                drain(gsems[cc], gval[cc])

                @pl.when(jo > 0)
                def _():
                    drain(ssems[cc], mval[cc])

                def scale(k8, _):
                    sl = pl.ds(k8 * L, L)
                    ew16 = ewv[j, sl]
                    for f in range(NF):
                        mval[cc][f, sl] = gval[cc][f, sl] * ew16
                    return 0

                lax.fori_loop(0, MB // L, scale, 0)

                @pl.when(j + RING < NB)
                def _():
                    fire_gathers(j + RING, cc)

                for f in range(NF):
                    pltpu.async_copy(mval[cc].at[f], accs[f].at[idxc.at[j]],
                                     ssems[cc], add=True)
            return carry

        lax.fori_loop(0, NB // RING, outer, 0)
        for cc in range(RING):
            drain(ssems[cc], mval[cc])
        plsc.subcore_barrier()

        for f in range(NF):
            @pl.when(s == f)
            def _(f=f):
                pltpu.sync_copy(accs[f], out_h.at[c, 0, pl.ds(f * NPAD, NPAD)])

    out = k(rows3, cols3, ew3, tab, zer)
    return out.reshape(NC, NF, NPAD)


def _tc_pre(x, W1, degp):
    def body(x_ref, w_ref, dp_ref, g_ref, dinv_ref):
        dp = dp_ref[...]
        deg = 1.0 + dp[0, 0, :N] + dp[1, 0, :N]
        dinv = lax.rsqrt(deg)[:, None]
        h = jnp.dot(x_ref[...], w_ref[...],
                    preferred_element_type=jnp.float32)
        g = (h * dinv).T
        g_ref[...] = jnp.concatenate(
            [g, jnp.zeros((16, NPAD - N), jnp.float32)], axis=1)
        dinv_ref[...] = dinv

    return pl.pallas_call(
        body,
        out_shape=(jax.ShapeDtypeStruct((16, NPAD), jnp.float32),
                   jax.ShapeDtypeStruct((N, 1), jnp.float32)),
    )(x, W1, degp)


def _tc_mid(agg, gv, dinv, bias, gam, bet, Wn, d_in, d_out):
    def body(a_ref, g_ref, di_ref, b_ref, ga_ref, be_ref, w_ref, o_ref):
        a = (a_ref[0] + a_ref[1] + g_ref[...])[:, :N].T
        dinv = di_ref[...]
        out = dinv * a + b_ref[...]
        bn = ga_ref[...] * (out * BNS) + be_ref[...]
        r = jnp.maximum(bn, 0.0)
        h = jnp.dot(r, w_ref[...], preferred_element_type=jnp.float32)
        g_next = (h * dinv).T
        o_ref[...] = jnp.concatenate(
            [g_next, jnp.zeros((d_out, NPAD - N), jnp.float32)], axis=1)

    return pl.pallas_call(
        body,
        out_shape=jax.ShapeDtypeStruct((d_out, NPAD), jnp.float32),
    )(agg, gv, dinv, bias, gam, bet, Wn)


def _tc_post(aggp, gv, dinv, bias, gam, bet, Wl, bl):
    def body(a_ref, g_ref, di_ref, b_ref, ga_ref, be_ref, wl_ref, bl_ref,
             o_ref):
        a = (a_ref[0] + a_ref[1] + g_ref[...])[:, :N].T
        out = di_ref[...] * a + b_ref[...]
        bn = ga_ref[...] * (out * BNS) + be_ref[...]
        z = bn * wl_ref[...] + bl_ref[...]
        o_ref[...] = jax.nn.sigmoid(z)

    return pl.pallas_call(
        body,
        out_shape=jax.ShapeDtypeStruct((N, 1), jnp.float32),
    )(aggp, gv, dinv, bias, gam, bet, Wl, bl)


def kernel(x, edge_index, edge_attr, W1, b1, g1, be1, W2, b2, g2, be2,
           W3, b3, g3, be3, Wl, bl):
    row = edge_index[0].astype(jnp.int32)
    col = edge_index[1].astype(jnp.int32)
    ew = edge_attr.astype(jnp.float32)
    pad = EPAD - E
    rows3 = jnp.pad(row, (0, pad)).reshape(NW, NB, MB)
    cols3 = jnp.pad(col, (0, pad)).reshape(NW, NB, MB)
    ew3 = jnp.pad(ew, (0, pad)).reshape(NW, NB, MB)
    zer16 = jnp.zeros((16 * NPAD,), jnp.float32)
    ones_t = jnp.ones((NPAD,), jnp.float32)

    degp = _sc_agg(rows3, cols3, ew3, ones_t, zer16[:NPAD], 1)

    g1t, dinv = _tc_pre(x, W1, degp)
    ag1 = _sc_agg(rows3, cols3, ew3, g1t.reshape(-1), zer16, 16)
    g2t = _tc_mid(ag1, g1t, dinv, b1.reshape(1, 16), g1.reshape(1, 16),
                  be1.reshape(1, 16), W2, 16, 4)

    ag2 = _sc_agg(rows3, cols3, ew3, g2t.reshape(-1), zer16[:4 * NPAD], 4)
    g3t = _tc_mid(ag2, g2t, dinv, b2.reshape(1, 4), g2.reshape(1, 4),
                  be2.reshape(1, 4), W3, 4, 1)

    ag3 = _sc_agg(rows3, cols3, ew3, g3t.reshape(-1), zer16[:NPAD], 1)
    y = _tc_post(ag3, g3t, dinv, b3.reshape(1, 1), g3.reshape(1, 1),
                 be3.reshape(1, 1), Wl, bl.reshape(1, 1))
    return y
